# baseline (speedup 1.0000x reference)
import jax
import jax.numpy as jnp
from jax.experimental import pallas as pl

K = 20
EPS = 1e-5


def _knn(x, k):
    inner = -2.0 * jnp.matmul(jnp.swapaxes(x, 2, 1), x)
    xx = jnp.sum(x ** 2, axis=1, keepdims=True)
    pairwise = -xx - inner - jnp.swapaxes(xx, 2, 1)
    return jax.lax.top_k(pairwise, k)[1]


def _graph_feature(x, k=K):
    B, C, N = x.shape
    idx = _knn(x, k)
    x_t = jnp.swapaxes(x, 2, 1)
    feature = jax.vmap(lambda xt, id_: jnp.take(xt, id_, axis=0))(x_t, idx)
    center = jnp.broadcast_to(x_t[:, :, None, :], (B, N, k, C))
    out = jnp.concatenate((feature - center, center), axis=3)
    return jnp.transpose(out, (0, 3, 1, 2))


def _conv(x, W):
    return jnp.einsum('bcnk,oc->bonk', x, W)


def _bn2d(x, g, b):
    mean = jnp.mean(x, axis=(0, 2, 3), keepdims=True)
    var = jnp.var(x, axis=(0, 2, 3), keepdims=True)
    xn = (x - mean) / jnp.sqrt(var + EPS)
    return xn * g[None, :, None, None] + b[None, :, None, None]


def _bn1d(x, g, b):
    mean = jnp.mean(x, axis=(0, 2), keepdims=True)
    var = jnp.var(x, axis=(0, 2), keepdims=True)
    xn = (x - mean) / jnp.sqrt(var + EPS)
    return xn * g[None, :, None] + b[None, :, None]


def _lrelu(x):
    return jnp.where(x > 0, x, 0.1 * x)


def kernel(x, W1, W2, W3, W4, W5, W6, g1, b1, g2, b2, g3, b3, g4, b4, g5, b5, g6, b6):
    h = _graph_feature(jnp.swapaxes(x, 2, 1))
    h = _lrelu(_bn2d(_conv(h, W1), g1, b1))
    h = _lrelu(_bn2d(_conv(h, W2), g2, b2))
    x1 = jnp.max(h, axis=-1)
    h = _graph_feature(x1)
    h = _lrelu(_bn2d(_conv(h, W3), g3, b3))
    h = _lrelu(_bn2d(_conv(h, W4), g4, b4))
    x2 = jnp.max(h, axis=-1)
    h = _graph_feature(x2)
    h = _lrelu(_bn2d(_conv(h, W5), g5, b5))
    x3 = jnp.max(h, axis=-1)
    h = jnp.concatenate((x1, x2, x3), axis=1)
    out = _lrelu(_bn1d(jnp.einsum('bcn,oc->bon', h, W6), g6, b6))
    return out


# speedup vs baseline: 9.8374x; 9.8374x over previous
import functools

import jax
import jax.numpy as jnp
from jax import lax
from jax.experimental import pallas as pl
from jax.experimental.pallas import tpu as pltpu
from jax.experimental.pallas import tpu_sc as plsc

K = 20
EPS = 1e-5
B, N = 8, 2048
BN = B * N
DP = 128
NEG = -3.0e38


def _lrelu(v):
    return jnp.where(v > 0, v, 0.1 * v)


def _knn_stage(xs, st, *, k, tn, apply_affine, interpret=False):
    b_, n_, cdim = xs.shape
    nt = n_ // tn

    def body(xs_ref, st_ref, idx_ref, xc_ref):
        b = pl.program_id(0)
        t = pl.program_id(1)
        xall = xs_ref[0]
        rows = xs_ref[0, pl.ds(t * tn, tn), :]
        if apply_affine:
            xall = _lrelu(xall * st_ref[0, :][None, :] + st_ref[1, :][None, :])
            rows = _lrelu(rows * st_ref[0, :][None, :] + st_ref[1, :][None, :])
        g = lax.dot_general(rows, xall, (((1,), (1,)), ((), ())),
                            preferred_element_type=jnp.float32)
        nn = jnp.sum(xall * xall, axis=1)
        ntile = jnp.sum(rows * rows, axis=1)
        p = 2.0 * g - ntile[:, None] - nn[None, :]
        iota = lax.broadcasted_iota(jnp.int32, (tn, n_), 1)
        base = b * n_
        for j in range(k):
            m = jnp.max(p, axis=1)
            eq = p >= m[:, None]
            pos = jnp.min(jnp.where(eq, iota, n_), axis=1)
            idx_ref[j, :] = pos + base
            p = jnp.where(iota == pos[:, None], NEG, p)
        xc_ref[0] = jnp.pad(rows, ((0, 0), (0, DP - cdim)))

    return pl.pallas_call(
        body,
        grid=(b_, nt),
        in_specs=[
            pl.BlockSpec((1, n_, cdim), lambda b, t: (b, 0, 0)),
            pl.BlockSpec((8, cdim), lambda b, t: (0, 0)),
        ],
        out_specs=[
            pl.BlockSpec((k, tn), lambda b, t: (0, b * (n_ // tn) + t)),
            pl.BlockSpec((1, tn, DP), lambda b, t: (b, t, 0)),
        ],
        out_shape=[
            jax.ShapeDtypeStruct((k, b_ * n_), jnp.int32),
            jax.ShapeDtypeStruct((b_, n_, DP), jnp.float32),
        ],
        interpret=interpret,
    )(xs, st)


def _sc_gather(table, idx_flat, d):
    r = idx_flat.shape[0]
    info = plsc.get_sparse_core_info()
    nc, ns = info.num_cores, info.num_subcores
    nw = nc * ns
    pw = r // nw
    gsz = 128
    nbuf = 4
    nchunk = pw // gsz
    mesh = plsc.VectorSubcoreMesh(core_axis_name="c", subcore_axis_name="s")

    @functools.partial(
        pl.kernel,
        mesh=mesh,
        out_type=jax.ShapeDtypeStruct((r, d), jnp.float32),
        scratch_types=[
            pltpu.VMEM((pw,), jnp.int32),
            [pltpu.VMEM((gsz, d), jnp.float32)] * nbuf,
            [pltpu.SemaphoreType.DMA] * nbuf,
        ],
    )
    def gather_kernel(table_hbm, idx_hbm, out_hbm, idx_v, bufs, sems):
        wid = lax.axis_index("s") * nc + lax.axis_index("c")
        base = pl.multiple_of(wid * pw, gsz)
        pltpu.sync_copy(idx_hbm.at[pl.ds(base, pw)], idx_v)

        def outer(o, _):
            g0 = pl.multiple_of(o * nbuf * gsz, gsz)
            for s in range(nbuf):
                off = pl.multiple_of(g0 + s * gsz, gsz)
                pltpu.make_async_copy(
                    table_hbm.at[idx_v.at[pl.ds(off, gsz)]], bufs[s], sems[s]
                ).start()
            for s in range(nbuf):
                off = pl.multiple_of(g0 + s * gsz, gsz)
                pltpu.make_async_copy(
                    table_hbm.at[idx_v.at[pl.ds(off, gsz)]], bufs[s], sems[s]
                ).wait()
                pltpu.sync_copy(bufs[s], out_hbm.at[pl.ds(base + off, gsz)])
            return 0

        lax.fori_loop(0, nchunk // nbuf, outer, 0)

    return gather_kernel(table, idx_flat)


def _edge_features(g_ref, c_ref, k, tp):
    g = g_ref[:, :, pl.ds(0, 64)]
    ctr = c_ref[:, pl.ds(0, 64)][None, :, :]
    e = jnp.concatenate([g - ctr, jnp.broadcast_to(ctr, g.shape)], axis=2)
    return e.reshape(k * tp, DP)


def _stage_stats(g3, ctr, wcat, *, k, tp, interpret=False):
    _, bn, _ = g3.shape
    d = wcat.shape[1]

    def body(g_ref, c_ref, w_ref, s_ref):
        i = pl.program_id(0)
        e = _edge_features(g_ref, c_ref, k, tp)
        h1 = jnp.dot(e, w_ref[...], preferred_element_type=jnp.float32)
        s = jnp.sum(h1, axis=0)
        sq = jnp.sum(h1 * h1, axis=0)
        upd = jnp.concatenate([s[None], sq[None], jnp.zeros((6, d), jnp.float32)], axis=0)

        @pl.when(i == 0)
        def _():
            s_ref[...] = jnp.zeros_like(s_ref)

        s_ref[...] += upd

    return pl.pallas_call(
        body,
        grid=(bn // tp,),
        in_specs=[
            pl.BlockSpec((k, tp, DP), lambda i: (0, i, 0)),
            pl.BlockSpec((tp, DP), lambda i: (i, 0)),
            pl.BlockSpec((DP, d), lambda i: (0, 0)),
        ],
        out_specs=pl.BlockSpec((8, d), lambda i: (0, 0)),
        out_shape=jax.ShapeDtypeStruct((8, d), jnp.float32),
        interpret=interpret,
    )(g3, ctr, wcat)


def _stage_main(g3, ctr, wcat, st1, w2t, *, k, tp, interpret=False):
    _, bn, _ = g3.shape
    d = wcat.shape[1]
    d2 = w2t.shape[1]

    def body(g_ref, c_ref, w_ref, st_ref, w2_ref, mx_ref, s_ref):
        i = pl.program_id(0)
        e = _edge_features(g_ref, c_ref, k, tp)
        h1 = jnp.dot(e, w_ref[...], preferred_element_type=jnp.float32)
        a = _lrelu(h1 * st_ref[0, :][None, :] + st_ref[1, :][None, :])
        h2 = jnp.dot(a, w2_ref[...], preferred_element_type=jnp.float32)
        s = jnp.sum(h2, axis=0)
        sq = jnp.sum(h2 * h2, axis=0)
        mx_ref[...] = jnp.max(h2.reshape(k, tp, d2), axis=0)
        upd = jnp.concatenate([s[None], sq[None], jnp.zeros((6, d2), jnp.float32)], axis=0)

        @pl.when(i == 0)
        def _():
            s_ref[...] = jnp.zeros_like(s_ref)

        s_ref[...] += upd

    return pl.pallas_call(
        body,
        grid=(bn // tp,),
        in_specs=[
            pl.BlockSpec((k, tp, DP), lambda i: (0, i, 0)),
            pl.BlockSpec((tp, DP), lambda i: (i, 0)),
            pl.BlockSpec((DP, d), lambda i: (0, 0)),
            pl.BlockSpec((8, d), lambda i: (0, 0)),
            pl.BlockSpec((d, d2), lambda i: (0, 0)),
        ],
        out_specs=[
            pl.BlockSpec((tp, d2), lambda i: (i, 0)),
            pl.BlockSpec((8, d2), lambda i: (0, 0)),
        ],
        out_shape=[
            jax.ShapeDtypeStruct((bn, d2), jnp.float32),
            jax.ShapeDtypeStruct((8, d2), jnp.float32),
        ],
        interpret=interpret,
    )(g3, ctr, wcat, st1, w2t)


def _stage3_maxstats(g3, ctr, wcat, *, k, tp, interpret=False):
    _, bn, _ = g3.shape
    d = wcat.shape[1]

    def body(g_ref, c_ref, w_ref, mx_ref, s_ref):
        i = pl.program_id(0)
        e = _edge_features(g_ref, c_ref, k, tp)
        h1 = jnp.dot(e, w_ref[...], preferred_element_type=jnp.float32)
        s = jnp.sum(h1, axis=0)
        sq = jnp.sum(h1 * h1, axis=0)
        mx_ref[...] = jnp.max(h1.reshape(k, tp, d), axis=0)
        upd = jnp.concatenate([s[None], sq[None], jnp.zeros((6, d), jnp.float32)], axis=0)

        @pl.when(i == 0)
        def _():
            s_ref[...] = jnp.zeros_like(s_ref)

        s_ref[...] += upd

    return pl.pallas_call(
        body,
        grid=(bn // tp,),
        in_specs=[
            pl.BlockSpec((k, tp, DP), lambda i: (0, i, 0)),
            pl.BlockSpec((tp, DP), lambda i: (i, 0)),
            pl.BlockSpec((DP, d), lambda i: (0, 0)),
        ],
        out_specs=[
            pl.BlockSpec((tp, d), lambda i: (i, 0)),
            pl.BlockSpec((8, d), lambda i: (0, 0)),
        ],
        out_shape=[
            jax.ShapeDtypeStruct((bn, d), jnp.float32),
            jax.ShapeDtypeStruct((8, d), jnp.float32),
        ],
        interpret=interpret,
    )(g3, ctr, wcat)


def _final_matmul(m1, st2, m2, st4, m3, st5, w6t, *, tp, interpret=False):
    bn = m1.shape[0]
    dco = w6t.shape[1]

    def body(m1_ref, s2_ref, m2_ref, s4_ref, m3_ref, s5_ref, w_ref, o_ref, s_ref):
        i = pl.program_id(0)
        x1 = _lrelu(m1_ref[...] * s2_ref[0, :][None, :] + s2_ref[1, :][None, :])
        x2 = _lrelu(m2_ref[...] * s4_ref[0, :][None, :] + s4_ref[1, :][None, :])
        x3 = _lrelu(m3_ref[...] * s5_ref[0, :][None, :] + s5_ref[1, :][None, :])
        h = jnp.concatenate([x1, x2, x3], axis=1)
        o = jnp.dot(h, w_ref[...], preferred_element_type=jnp.float32)
        o_ref[...] = o
        s = jnp.sum(o, axis=0)
        sq = jnp.sum(o * o, axis=0)
        upd = jnp.concatenate([s[None], sq[None], jnp.zeros((6, dco), jnp.float32)], axis=0)

        @pl.when(i == 0)
        def _():
            s_ref[...] = jnp.zeros_like(s_ref)

        s_ref[...] += upd

    return pl.pallas_call(
        body,
        grid=(bn // tp,),
        in_specs=[
            pl.BlockSpec((tp, 64), lambda i: (i, 0)),
            pl.BlockSpec((8, 64), lambda i: (0, 0)),
            pl.BlockSpec((tp, 64), lambda i: (i, 0)),
            pl.BlockSpec((8, 64), lambda i: (0, 0)),
            pl.BlockSpec((tp, 128), lambda i: (i, 0)),
            pl.BlockSpec((8, 128), lambda i: (0, 0)),
            pl.BlockSpec((256, dco), lambda i: (0, 0)),
        ],
        out_specs=[
            pl.BlockSpec((tp, dco), lambda i: (i, 0)),
            pl.BlockSpec((8, dco), lambda i: (0, 0)),
        ],
        out_shape=[
            jax.ShapeDtypeStruct((bn, dco), jnp.float32),
            jax.ShapeDtypeStruct((8, dco), jnp.float32),
        ],
        interpret=interpret,
    )(m1, st2, m2, st4, m3, st5, w6t)


def _final_norm(opre, st6, *, b, n, tp, interpret=False):
    dco = opre.shape[1]
    nt = n // tp

    def body(o_ref, st_ref, out_ref):
        o = _lrelu(o_ref[...] * st_ref[0, :][None, :] + st_ref[1, :][None, :])
        out_ref[0] = o.T

    return pl.pallas_call(
        body,
        grid=(b, nt),
        in_specs=[
            pl.BlockSpec((tp, dco), lambda bb, t: (bb * (n // tp) + t, 0)),
            pl.BlockSpec((8, dco), lambda bb, t: (0, 0)),
        ],
        out_specs=pl.BlockSpec((1, dco, tp), lambda bb, t: (bb, 0, t)),
        out_shape=jax.ShapeDtypeStruct((b, dco, n), jnp.float32),
        interpret=interpret,
    )(opre, st6)


def _affine_from_sums(sums, g, bparam, cnt):
    mean = sums[0] / cnt
    var = sums[1] / cnt - mean * mean
    scale = g / jnp.sqrt(var + EPS)
    shift = bparam - mean * scale
    d = scale.shape[0]
    return jnp.concatenate(
        [scale[None], shift[None], jnp.zeros((6, d), jnp.float32)], axis=0)


def _build_wcat(w, c):
    d = w.shape[0]
    wl = w[:, :c].T
    wr = w[:, c:].T
    top = jnp.concatenate([wl, jnp.zeros((64 - c, d), jnp.float32)], axis=0)
    bot = jnp.concatenate([wr, jnp.zeros((64 - c, d), jnp.float32)], axis=0)
    return jnp.concatenate([top, bot], axis=0)


def kernel(x, W1, W2, W3, W4, W5, W6, g1, b1, g2, b2, g3, b3, g4, b4, g5, b5, g6, b6):
    return _pipeline(x, W1, W2, W3, W4, W5, W6, g1, b1, g2, b2, g3, b3, g4,
                     b4, g5, b5, g6, b6, b=B, n=N)


def _pipeline(x, W1, W2, W3, W4, W5, W6, g1, b1, g2, b2, g3, b3, g4, b4, g5,
              b5, g6, b6, *, b, n, tn=256, tp=512, interpret=False):
    B, N, BN = b, n, b * n
    k = K
    cnt2d = float(B * N * k)
    cnt1d = float(B * N)

    xs1 = jnp.concatenate([x, jnp.zeros((B, N, 5), jnp.float32)], axis=2)
    st_id = jnp.zeros((8, 8), jnp.float32)
    idx1, tab1 = _knn_stage(xs1, st_id, k=k, tn=tn, apply_affine=False,
                            interpret=interpret)
    gv1 = _sc_gather(tab1.reshape(BN, DP), idx1.reshape(-1), DP)
    g31 = gv1.reshape(k, BN, DP)
    ctr1 = tab1.reshape(BN, DP)
    wcat1 = _build_wcat(W1, 3)
    sums1 = _stage_stats(g31, ctr1, wcat1, k=k, tp=tp, interpret=interpret)
    st_bn1 = _affine_from_sums(sums1, g1, b1, cnt2d)
    maxh1, sums2 = _stage_main(g31, ctr1, wcat1, st_bn1, W2.T, k=k, tp=tp,
                               interpret=interpret)
    st_bn2 = _affine_from_sums(sums2, g2, b2, cnt2d)

    idx2, tab2 = _knn_stage(maxh1.reshape(B, N, 64), st_bn2[:, :64], k=k,
                            tn=tn, apply_affine=True, interpret=interpret)
    gv2 = _sc_gather(tab2.reshape(BN, DP), idx2.reshape(-1), DP)
    g32 = gv2.reshape(k, BN, DP)
    ctr2 = tab2.reshape(BN, DP)
    wcat3 = _build_wcat(W3, 64)
    sums3 = _stage_stats(g32, ctr2, wcat3, k=k, tp=tp, interpret=interpret)
    st_bn3 = _affine_from_sums(sums3, g3, b3, cnt2d)
    maxh2, sums4 = _stage_main(g32, ctr2, wcat3, st_bn3, W4.T, k=k, tp=tp,
                               interpret=interpret)
    st_bn4 = _affine_from_sums(sums4, g4, b4, cnt2d)

    idx3, tab3 = _knn_stage(maxh2.reshape(B, N, 64), st_bn4[:, :64], k=k,
                            tn=tn, apply_affine=True, interpret=interpret)
    gv3 = _sc_gather(tab3.reshape(BN, DP), idx3.reshape(-1), DP)
    g33 = gv3.reshape(k, BN, DP)
    ctr3 = tab3.reshape(BN, DP)
    wcat5 = _build_wcat(W5, 64)
    maxh3, sums5 = _stage3_maxstats(g33, ctr3, wcat5, k=k, tp=tp,
                                    interpret=interpret)
    st_bn5 = _affine_from_sums(sums5, g5, b5, cnt2d)

    opre, sums6 = _final_matmul(maxh1, st_bn2, maxh2, st_bn4, maxh3, st_bn5,
                                W6.T, tp=tp, interpret=interpret)
    st_bn6 = _affine_from_sums(sums6, g6, b6, cnt1d)
    return _final_norm(opre, st_bn6, b=B, n=N, tp=tp, interpret=interpret)
